# Initial kernel scaffold; baseline (speedup 1.0000x reference)
import jax
import jax.numpy as jnp
from jax.experimental import pallas as pl


def kernel(out):
    raise NotImplementedError("write your pallas kernel here")


# speedup vs baseline: 2.3230x; 2.3230x over previous
import jax
import jax.numpy as jnp
from jax.experimental import pallas as pl
from jax.experimental.pallas import tpu as pltpu

_NPG = 256
_K = _NPG // 2 + 1


def _body(x_ref, o_ref):
    x = x_ref[...]
    b = jax.lax.bitcast_convert_type(x, jnp.int32)
    key = jnp.where(b < 0, b ^ jnp.int32(0x7FFFFFFF), b)

    c0 = jnp.sum((key >= 0).astype(jnp.int32), axis=1, keepdims=True)
    t = jnp.where(c0 >= _K, jnp.int32(0), jnp.int32(-0x80000000))

    def pass_fn(i, t):
        bit = jnp.int32(1) << (jnp.int32(30) - i)
        t_try = t + bit
        c = jnp.sum((key >= t_try).astype(jnp.int32), axis=1, keepdims=True)
        return jnp.where(c >= _K, t_try, t)

    t = jax.lax.fori_loop(0, 31, pass_fn, t)

    midb = jnp.where(t < 0, t ^ jnp.int32(0x7FFFFFFF), t)
    midv = jax.lax.bitcast_convert_type(midb, jnp.float32)

    m = jnp.max(x, axis=1, keepdims=True)
    e = jnp.exp(x - m)
    s = jnp.sum(e, axis=1, keepdims=True)
    o_ref[...] = jnp.where(x > midv, e / s, 0.0)


def kernel(out):
    rows = out.size // _NPG
    x = out.reshape(rows, _NPG)
    br = 512
    res = pl.pallas_call(
        _body,
        grid=(rows // br,),
        in_specs=[pl.BlockSpec((br, _NPG), lambda i: (i, 0))],
        out_specs=pl.BlockSpec((br, _NPG), lambda i: (i, 0)),
        out_shape=jax.ShapeDtypeStruct((rows, _NPG), jnp.float32),
    )(x)
    return res


# speedup vs baseline: 3.7561x; 1.6169x over previous
import jax
import jax.numpy as jnp
from jax.experimental import pallas as pl
from jax.experimental.pallas import tpu as pltpu

_NPG = 256
_K = _NPG // 2 + 1


def _body(x_ref, o_ref):
    x = x_ref[...]
    b = jax.lax.bitcast_convert_type(x, jnp.int32)
    key = jnp.where(b < 0, b ^ jnp.int32(0x7FFFFFFF), b)

    c0 = jnp.sum((key >= 0).astype(jnp.int32), axis=1, keepdims=True)
    t = jnp.where(c0 >= _K, jnp.int32(0), jnp.int32(-0x80000000))

    for i in range(30, -1, -1):
        t_try = t + jnp.int32(1 << i)
        c = jnp.sum((key >= t_try).astype(jnp.int32), axis=1, keepdims=True)
        t = jnp.where(c >= _K, t_try, t)

    midb = jnp.where(t < 0, t ^ jnp.int32(0x7FFFFFFF), t)
    midv = jax.lax.bitcast_convert_type(midb, jnp.float32)

    m = jnp.max(x, axis=1, keepdims=True)
    e = jnp.exp(x - m)
    s = jnp.sum(e, axis=1, keepdims=True)
    o_ref[...] = jnp.where(x > midv, e / s, 0.0)


def kernel(out):
    rows = out.size // _NPG
    x = out.reshape(rows, _NPG)
    br = 512
    res = pl.pallas_call(
        _body,
        grid=(rows // br,),
        in_specs=[pl.BlockSpec((br, _NPG), lambda i: (i, 0))],
        out_specs=pl.BlockSpec((br, _NPG), lambda i: (i, 0)),
        out_shape=jax.ShapeDtypeStruct((rows, _NPG), jnp.float32),
    )(x)
    return res


# speedup vs baseline: 5.2075x; 1.3864x over previous
import jax
import jax.numpy as jnp
from jax.experimental import pallas as pl
from jax.experimental.pallas import tpu as pltpu

_NPG = 256
_K = _NPG // 2 + 1


def _body(x_ref, o_ref):
    x = x_ref[...]
    xt = x.T
    b = jax.lax.bitcast_convert_type(xt, jnp.int32)
    key = jnp.where(b < 0, b ^ jnp.int32(0x7FFFFFFF), b)

    c0 = jnp.sum((key >= 0).astype(jnp.int32), axis=0, keepdims=True)
    t = jnp.where(c0 >= _K, jnp.int32(0), jnp.int32(-0x80000000))

    for i in range(30, -1, -1):
        t_try = t + jnp.int32(1 << i)
        c = jnp.sum((key >= t_try).astype(jnp.int32), axis=0, keepdims=True)
        t = jnp.where(c >= _K, t_try, t)

    midb = jnp.where(t < 0, t ^ jnp.int32(0x7FFFFFFF), t)
    midv = jax.lax.bitcast_convert_type(midb, jnp.float32)

    m = jnp.max(xt, axis=0, keepdims=True)
    e = jnp.exp(xt - m)
    s = jnp.sum(e, axis=0, keepdims=True)
    ot = jnp.where(xt > midv, e / s, 0.0)
    o_ref[...] = ot.T


def kernel(out):
    rows = out.size // _NPG
    x = out.reshape(rows, _NPG)
    br = 512
    res = pl.pallas_call(
        _body,
        grid=(rows // br,),
        in_specs=[pl.BlockSpec((br, _NPG), lambda i: (i, 0))],
        out_specs=pl.BlockSpec((br, _NPG), lambda i: (i, 0)),
        out_shape=jax.ShapeDtypeStruct((rows, _NPG), jnp.float32),
    )(x)
    return res


# speedup vs baseline: 7.7956x; 1.4970x over previous
import jax
import jax.numpy as jnp
from jax.experimental import pallas as pl
from jax.experimental.pallas import tpu as pltpu

_NPG = 256
_K = _NPG // 2 + 1


def _body(x_ref, o_ref):
    x = x_ref[...]
    xt = x.T
    b = jax.lax.bitcast_convert_type(xt, jnp.int32)
    key = jnp.where(b < 0, b ^ jnp.int32(0x7FFFFFFF), b)

    hi = (key >> 16).astype(jnp.int16)
    lo = ((key & 0xFFFF) - 32768).astype(jnp.int16)

    def csum(m):
        s = m[0:128] + m[128:256]
        s = s[0:64] + s[64:128]
        s = s[0:32] + s[32:64]
        s = s[0:16] + s[16:32]
        return jnp.sum(s.astype(jnp.int32), axis=0, keepdims=True)

    one16 = jnp.int16(1)
    zero16 = jnp.int16(0)

    c0 = csum(jnp.where(hi >= 0, one16, zero16))
    th = jnp.where(c0 >= _K, jnp.int32(0), jnp.int32(-32768))
    for i in range(14, -1, -1):
        t_try = th + jnp.int32(1 << i)
        c = csum(jnp.where(hi >= t_try.astype(jnp.int16), one16, zero16))
        th = jnp.where(c >= _K, t_try, th)

    th16 = th.astype(jnp.int16)
    cgt = csum(jnp.where(hi > th16, one16, zero16))
    kp = _K - cgt
    lo_eff = jnp.where(hi == th16, lo, jnp.int16(-32768))
    cl0 = csum(jnp.where(lo_eff >= 0, one16, zero16))
    tl = jnp.where(cl0 >= kp, jnp.int32(0), jnp.int32(-32768))
    for i in range(14, -1, -1):
        t_try = tl + jnp.int32(1 << i)
        c = csum(jnp.where(lo_eff >= t_try.astype(jnp.int16), one16, zero16))
        tl = jnp.where(c >= kp, t_try, tl)

    t = (th << 16) | ((tl + 32768) & 0xFFFF)
    midb = jnp.where(t < 0, t ^ jnp.int32(0x7FFFFFFF), t)
    midv = jax.lax.bitcast_convert_type(midb, jnp.float32)

    m = jnp.max(xt, axis=0, keepdims=True)
    e = jnp.exp(xt - m)
    s = jnp.sum(e, axis=0, keepdims=True)
    ot = jnp.where(xt > midv, e * (1.0 / s), 0.0)
    o_ref[...] = ot.T


def kernel(out):
    rows = out.size // _NPG
    x = out.reshape(rows, _NPG)
    br = 512
    res = pl.pallas_call(
        _body,
        grid=(rows // br,),
        in_specs=[pl.BlockSpec((br, _NPG), lambda i: (i, 0))],
        out_specs=pl.BlockSpec((br, _NPG), lambda i: (i, 0)),
        out_shape=jax.ShapeDtypeStruct((rows, _NPG), jnp.float32),
    )(x)
    return res


# speedup vs baseline: 8.9970x; 1.1541x over previous
import jax
import jax.numpy as jnp
from jax.experimental import pallas as pl
from jax.experimental.pallas import tpu as pltpu

_NPG = 256
_K = _NPG // 2 + 1


def _body(x_ref, o_ref):
    x = x_ref[...]
    xt = x.T
    b = jax.lax.bitcast_convert_type(xt, jnp.int32)
    key = jnp.where(b < 0, b ^ jnp.int32(0x7FFFFFFF), b)

    hi = (key >> 16).astype(jnp.int16)
    lo = ((key & 0xFFFF) - 32768).astype(jnp.int16)

    def csum(m):
        s = m[0:128] + m[128:256]
        s = s[0:64] + s[64:128]
        s = s[0:32] + s[32:64]
        s = s[0:16] + s[16:32]
        return jnp.sum(s.astype(jnp.int32), axis=0, keepdims=True)

    one16 = jnp.int16(1)
    zero16 = jnp.int16(0)

    c0 = csum(jnp.where(hi >= 0, one16, zero16))
    th = jnp.where(c0 >= _K, jnp.int32(0), jnp.int32(-32768))
    for i in range(14, -1, -1):
        t_try = th + jnp.int32(1 << i)
        c = csum(jnp.where(hi >= t_try.astype(jnp.int16), one16, zero16))
        th = jnp.where(c >= _K, t_try, th)

    th16 = th.astype(jnp.int16)
    cgt = csum(jnp.where(hi > th16, one16, zero16))
    kp = _K - cgt
    lo_eff = jnp.where(hi == th16, lo, jnp.int16(-32768))
    cl0 = csum(jnp.where(lo_eff >= 0, one16, zero16))
    tl = jnp.where(cl0 >= kp, jnp.int32(0), jnp.int32(-32768))
    for i in range(14, -1, -1):
        t_try = tl + jnp.int32(1 << i)
        c = csum(jnp.where(lo_eff >= t_try.astype(jnp.int16), one16, zero16))
        tl = jnp.where(c >= kp, t_try, tl)

    t = (th << 16) | ((tl + 32768) & 0xFFFF)
    midb = jnp.where(t < 0, t ^ jnp.int32(0x7FFFFFFF), t)
    midv = jax.lax.bitcast_convert_type(midb, jnp.float32)

    m = jnp.max(xt, axis=0, keepdims=True)
    e = jnp.exp(xt - m)
    s = jnp.sum(e, axis=0, keepdims=True)
    ot = jnp.where(xt > midv, e * (1.0 / s), 0.0)
    o_ref[...] = ot.T


def kernel(out):
    rows = out.size // _NPG
    x = out.reshape(rows, _NPG)
    br = 2048
    res = pl.pallas_call(
        _body,
        grid=(rows // br,),
        in_specs=[pl.BlockSpec((br, _NPG), lambda i: (i, 0))],
        out_specs=pl.BlockSpec((br, _NPG), lambda i: (i, 0)),
        out_shape=jax.ShapeDtypeStruct((rows, _NPG), jnp.float32),
    )(x)
    return res


# speedup vs baseline: 11.3562x; 1.2622x over previous
import jax
import jax.numpy as jnp
from jax.experimental import pallas as pl
from jax.experimental.pallas import tpu as pltpu

_NPG = 256
_K = _NPG // 2 + 1


def _body(x_ref, o_ref):
    kg = x_ref.shape[1] // _NPG
    xt = jnp.concatenate(
        [x_ref[:, j * _NPG:(j + 1) * _NPG].T for j in range(kg)], axis=1)
    b = jax.lax.bitcast_convert_type(xt, jnp.int32)
    key = jnp.where(b < 0, b ^ jnp.int32(0x7FFFFFFF), b)

    hi = (key >> 16).astype(jnp.int16)
    lo = ((key & 0xFFFF) - 32768).astype(jnp.int16)

    def csum(m):
        s = m[0:128] + m[128:256]
        s = s[0:64] + s[64:128]
        s = s[0:32] + s[32:64]
        s = s[0:16] + s[16:32]
        return jnp.sum(s.astype(jnp.int32), axis=0, keepdims=True)

    one16 = jnp.int16(1)
    zero16 = jnp.int16(0)

    c0 = csum(jnp.where(hi >= 0, one16, zero16))
    th = jnp.where(c0 >= _K, jnp.int32(0), jnp.int32(-32768))
    for i in range(14, -1, -1):
        t_try = th + jnp.int32(1 << i)
        c = csum(jnp.where(hi >= t_try.astype(jnp.int16), one16, zero16))
        th = jnp.where(c >= _K, t_try, th)

    th16 = th.astype(jnp.int16)
    cgt = csum(jnp.where(hi > th16, one16, zero16))
    kp = _K - cgt
    lo_eff = jnp.where(hi == th16, lo, jnp.int16(-32768))
    cl0 = csum(jnp.where(lo_eff >= 0, one16, zero16))
    tl = jnp.where(cl0 >= kp, jnp.int32(0), jnp.int32(-32768))
    for i in range(14, -1, -1):
        t_try = tl + jnp.int32(1 << i)
        c = csum(jnp.where(lo_eff >= t_try.astype(jnp.int16), one16, zero16))
        tl = jnp.where(c >= kp, t_try, tl)

    t = (th << 16) | ((tl + 32768) & 0xFFFF)
    midb = jnp.where(t < 0, t ^ jnp.int32(0x7FFFFFFF), t)
    midv = jax.lax.bitcast_convert_type(midb, jnp.float32)

    m = jnp.max(xt, axis=0, keepdims=True)
    e = jnp.exp(xt - m)
    s = jnp.sum(e, axis=0, keepdims=True)
    ot = jnp.where(xt > midv, e * (1.0 / s), 0.0)
    for j in range(kg):
        o_ref[:, j, :] = ot[:, j * 128:(j + 1) * 128].T


def kernel(out):
    nr, nc = out.shape
    ng = nc // _NPG
    kg = 16
    res3 = pl.pallas_call(
        _body,
        grid=(ng // kg,),
        in_specs=[pl.BlockSpec((nr, kg * _NPG), lambda i: (0, i))],
        out_specs=pl.BlockSpec((nr, kg, _NPG), lambda i: (0, i, 0)),
        out_shape=jax.ShapeDtypeStruct((nr, ng, _NPG), jnp.float32),
    )(out)
    return res3.reshape(nr * ng, _NPG)


# speedup vs baseline: 11.8924x; 1.0472x over previous
import jax
import jax.numpy as jnp
from jax.experimental import pallas as pl
from jax.experimental.pallas import tpu as pltpu

_NPG = 256
_K = _NPG // 2 + 1


def _body(x_ref, o_ref):
    kg = x_ref.shape[1] // _NPG
    xt = jnp.concatenate(
        [x_ref[:, j * _NPG:(j + 1) * _NPG].T for j in range(kg)], axis=1)
    b = jax.lax.bitcast_convert_type(xt, jnp.int32)
    key = jnp.where(b < 0, b ^ jnp.int32(0x7FFFFFFF), b)

    hi = (key >> 16).astype(jnp.int16)
    lo = ((key & 0xFFFF) - 32768).astype(jnp.int16)

    def csum(m):
        s = m[0:128] + m[128:256]
        s = s[0:64] + s[64:128]
        s = s[0:32] + s[32:64]
        s = s[0:16] + s[16:32]
        return jnp.sum(s.astype(jnp.int32), axis=0, keepdims=True)

    one16 = jnp.int16(1)
    zero16 = jnp.int16(0)

    c0 = csum(jnp.where(hi >= 0, one16, zero16))
    th = jnp.where(c0 >= _K, jnp.int32(0), jnp.int32(-32768))
    for i in range(14, -1, -1):
        t_try = th + jnp.int32(1 << i)
        c = csum(jnp.where(hi >= t_try.astype(jnp.int16), one16, zero16))
        th = jnp.where(c >= _K, t_try, th)

    th16 = th.astype(jnp.int16)
    cgt = csum(jnp.where(hi > th16, one16, zero16))
    kp = _K - cgt
    lo_eff = jnp.where(hi == th16, lo, jnp.int16(-32768))
    cl0 = csum(jnp.where(lo_eff >= 0, one16, zero16))
    tl = jnp.where(cl0 >= kp, jnp.int32(0), jnp.int32(-32768))
    for i in range(14, -1, -1):
        t_try = tl + jnp.int32(1 << i)
        c = csum(jnp.where(lo_eff >= t_try.astype(jnp.int16), one16, zero16))
        tl = jnp.where(c >= kp, t_try, tl)

    t = (th << 16) | ((tl + 32768) & 0xFFFF)
    midb = jnp.where(t < 0, t ^ jnp.int32(0x7FFFFFFF), t)
    midv = jax.lax.bitcast_convert_type(midb, jnp.float32)

    m = jnp.max(xt, axis=0, keepdims=True)
    e = jnp.exp(xt - m)
    s = jnp.sum(e, axis=0, keepdims=True)
    ot = jnp.where(xt > midv, e * (1.0 / s), 0.0)
    for j in range(kg):
        o_ref[:, j, :] = ot[:, j * 128:(j + 1) * 128].T


def kernel(out):
    nr, nc = out.shape
    ng = nc // _NPG
    kg = 8
    res3 = pl.pallas_call(
        _body,
        grid=(ng // kg,),
        in_specs=[pl.BlockSpec((nr, kg * _NPG), lambda i: (0, i))],
        out_specs=pl.BlockSpec((nr, kg, _NPG), lambda i: (0, i, 0)),
        out_shape=jax.ShapeDtypeStruct((nr, ng, _NPG), jnp.float32),
    )(out)
    return res3.reshape(nr * ng, _NPG)
